# baseline (speedup 1.0000x reference)
import functools

import jax
import jax.numpy as jnp
from jax import lax
from jax.experimental import pallas as pl
from jax.experimental.pallas import tpu as pltpu
from jax.experimental.pallas import tpu_sc as plsc

_BETA = 0.25


def _argmin_body(TN, GN, zt_ref, w2_ref, idx_ref, dsum_ref, wsq_s, zsq_s, rmin_s, rbase_s):
    i, j = pl.program_id(0), pl.program_id(1)
    nj = pl.num_programs(1)

    @pl.when(i == 0)
    def _():
        w2 = w2_ref[pl.ds(j * TN, TN), :]
        wsq_s[pl.ds(j * TN, TN), :] = 0.25 * jnp.sum(w2 * w2, axis=1, keepdims=True)

    zt = zt_ref[0]

    @pl.when(j == 0)
    def _():
        zsq_s[...] = jnp.sum(zt * zt, axis=0, keepdims=True)
        rmin_s[...] = jnp.full(rmin_s.shape, jnp.inf, jnp.float32)
        rbase_s[...] = jnp.zeros(rbase_s.shape, jnp.int32)

    zsq = zsq_s[...]
    rmin = rmin_s[...]
    rbase = rbase_s[...]
    for g in range(TN // GN):
        base = j * TN + g * GN
        wg = w2_ref[pl.ds(base, GN), :]
        mg = jax.lax.dot_general(wg, zt, (((1,), (0,)), ((), ())),
                                 preferred_element_type=jnp.float32)
        dg = (zsq + mg) + wsq_s[pl.ds(base, GN), :]
        for s in range(GN // 8):
            v = dg[s * 8:(s + 1) * 8, :]
            lt = v < rmin
            rbase = jnp.where(lt, base + s * 8, rbase)
            rmin = jnp.where(lt, v, rmin)
    rmin_s[...] = rmin
    rbase_s[...] = rbase

    @pl.when(jnp.logical_and(i == 0, j == 0))
    def _():
        dsum_ref[...] = jnp.zeros((1, 1), jnp.float32)

    @pl.when(j == nj - 1)
    def _():
        ix = rbase + jax.lax.broadcasted_iota(jnp.int32, rbase.shape, 0)
        lmin = jnp.min(rmin, axis=0, keepdims=True)
        cand = jnp.where(rmin == lmin, ix, jnp.int32(2**30))
        idx_ref[...] = jnp.min(cand, axis=0)
        dsum_ref[...] += jnp.sum(lmin).reshape(1, 1)


def _tc_argmin(z3d, W2, TM=512, TN=8192, GN=128, interpret=False):
    B, K, T = z3d.shape
    M = B * T
    N = W2.shape[0]
    grid = (M // TM, N // TN)
    ti = T // TM
    return pl.pallas_call(
        functools.partial(_argmin_body, TN, GN),
        grid=grid,
        in_specs=[
            pl.BlockSpec((1, K, TM), lambda i, j: (i // ti, 0, i % ti)),
            pl.BlockSpec((N, K), lambda i, j: (0, 0)),
        ],
        out_specs=[
            pl.BlockSpec((TM,), lambda i, j: (i,)),
            pl.BlockSpec((1, 1), lambda i, j: (0, 0)),
        ],
        out_shape=[
            jax.ShapeDtypeStruct((M,), jnp.int32),
            jax.ShapeDtypeStruct((1, 1), jnp.float32),
        ],
        scratch_shapes=[
            pltpu.VMEM((N, 1), jnp.float32),
            pltpu.VMEM((1, TM), jnp.float32),
            pltpu.VMEM((8, TM), jnp.float32),
            pltpu.VMEM((8, TM), jnp.int32),
        ],
        interpret=interpret,
    )(z3d, W2)


@functools.cache
def _sc_gather_hist(M, N, D):
    info = plsc.get_sparse_core_info()
    NC, NS, L = info.num_cores, info.num_subcores, info.num_lanes
    NW = NC * NS
    Bv = M // NW
    CH = 128
    n_ch = Bv // CH
    mesh = plsc.VectorSubcoreMesh(core_axis_name="c", subcore_axis_name="s")

    @functools.partial(
        pl.kernel, mesh=mesh,
        out_type=[jax.ShapeDtypeStruct((M, D), jnp.float32),
                  jax.ShapeDtypeStruct((NC, N), jnp.float32)],
        scratch_types=[
            pltpu.VMEM((Bv,), jnp.int32),
            pltpu.VMEM((CH, D), jnp.float32),
            pltpu.VMEM((CH, D), jnp.float32),
            pltpu.VMEM((Bv,), jnp.float32),
            pltpu.VMEM((N,), jnp.float32),
            pltpu.VMEM_SHARED((N,), jnp.float32),
            pltpu.SemaphoreType.DMA,
            pltpu.SemaphoreType.DMA,
        ],
    )
    def k(w_hbm, idx_hbm, out_hbm, hist_hbm,
          idx_v, bufa, bufb, ones_v, zeros_v, hist_sh, sema, semb):
        c = lax.axis_index("c")
        s = lax.axis_index("s")
        wid = s * NC + c
        base = wid * Bv
        pltpu.sync_copy(idx_hbm.at[pl.ds(base, Bv)], idx_v)

        bufs = [bufa, bufb]
        sems = [sema, semb]
        prev = pltpu.async_copy(w_hbm.at[idx_v.at[pl.ds(0, CH)]], bufa, sema)
        for ch in range(1, n_ch):
            cur = pltpu.async_copy(w_hbm.at[idx_v.at[pl.ds(ch * CH, CH)]],
                                   bufs[ch % 2], sems[ch % 2])
            prev.wait()
            pltpu.sync_copy(bufs[(ch - 1) % 2],
                            out_hbm.at[pl.ds(base + (ch - 1) * CH, CH)])
            prev = cur
        prev.wait()
        pltpu.sync_copy(bufs[(n_ch - 1) % 2],
                        out_hbm.at[pl.ds(base + (n_ch - 1) * CH, CH)])

        def fill_ones(t, carry):
            ones_v[pl.ds(t * L, L)] = jnp.full((L,), 1.0, jnp.float32)
            return carry
        lax.fori_loop(0, Bv // L, fill_ones, 0)

        @pl.when(s == 0)
        def _():
            def fill_zeros(t, carry):
                zeros_v[pl.ds(t * L, L)] = jnp.zeros((L,), jnp.float32)
                return carry
            lax.fori_loop(0, N // L, fill_zeros, 0)
            pltpu.sync_copy(zeros_v, hist_sh)

        plsc.subcore_barrier()
        for kk in range(n_ch):
            pltpu.sync_copy(ones_v.at[pl.ds(kk * CH, CH)],
                            hist_sh.at[idx_v.at[pl.ds(kk * CH, CH)]],
                            add=True)
        plsc.subcore_barrier()

        @pl.when(s == 0)
        def _():
            pltpu.sync_copy(hist_sh, hist_hbm.at[c])

    return k


def kernel(z, W):
    B, C, T = z.shape
    N = W.shape[0]
    n_tok = B * T
    half = B // 2
    W2 = W * -2.0
    sc = _sc_gather_hist(n_tok // 2, N, C)
    idx1, dsum1 = _tc_argmin(z[:half], W2)
    q1, h1 = sc(W, idx1)
    idx2, dsum2 = _tc_argmin(z[half:], W2)
    q2, h2 = sc(W, idx2)
    idx = jnp.concatenate([idx1, idx2])
    dsum = dsum1 + dsum2
    quantized = jnp.concatenate([q1, q2])
    counts = (h1[0] + h1[1]) + (h2[0] + h2[1])
    mse = dsum[0, 0] / (n_tok * C)
    codebook_loss = mse
    commitment_loss = mse
    vq_loss = codebook_loss + _BETA * commitment_loss
    quantized_out = jnp.transpose(quantized.reshape(B, T, C), (0, 2, 1))
    avg_probs = counts / n_tok
    perplexity = jnp.exp(-jnp.sum(avg_probs * jnp.log(avg_probs + 1e-10)))
    return (quantized_out, idx.reshape(B, T), vq_loss, codebook_loss,
            commitment_loss, perplexity)


# speedup vs baseline: 1.1151x; 1.1151x over previous
import functools

import jax
import jax.numpy as jnp
from jax import lax
from jax.experimental import pallas as pl
from jax.experimental.pallas import tpu as pltpu
from jax.experimental.pallas import tpu_sc as plsc

_BETA = 0.25


def _argmin_body(TN, GN, zt_ref, w2_ref, idx_ref, dsum_ref, wsq_s, zsq_s, rmin_s, rbase_s):
    i, j = pl.program_id(0), pl.program_id(1)
    nj = pl.num_programs(1)

    @pl.when(i == 0)
    def _():
        w2 = w2_ref[pl.ds(j * TN, TN), :]
        wsq_s[pl.ds(j * TN, TN), :] = 0.25 * jnp.sum(w2 * w2, axis=1, keepdims=True)

    zt = zt_ref[0]

    @pl.when(j == 0)
    def _():
        zsq_s[...] = jnp.sum(zt * zt, axis=0, keepdims=True)
        rmin_s[...] = jnp.full(rmin_s.shape, jnp.inf, jnp.float32)
        rbase_s[...] = jnp.zeros(rbase_s.shape, jnp.int32)

    zsq = zsq_s[...]
    rmin = rmin_s[...]
    rbase = rbase_s[...]
    for g in range(TN // GN):
        base = j * TN + g * GN
        wg = w2_ref[pl.ds(base, GN), :]
        mg = jax.lax.dot_general(wg, zt, (((1,), (0,)), ((), ())),
                                 preferred_element_type=jnp.float32)
        dg = (zsq + mg) + wsq_s[pl.ds(base, GN), :]
        for s in range(GN // 8):
            v = dg[s * 8:(s + 1) * 8, :]
            lt = v < rmin
            rbase = jnp.where(lt, base + s * 8, rbase)
            rmin = jnp.where(lt, v, rmin)
    rmin_s[...] = rmin
    rbase_s[...] = rbase

    @pl.when(jnp.logical_and(i == 0, j == 0))
    def _():
        dsum_ref[...] = jnp.zeros((1, 1), jnp.float32)

    @pl.when(j == nj - 1)
    def _():
        ix = rbase + jax.lax.broadcasted_iota(jnp.int32, rbase.shape, 0)
        lmin = jnp.min(rmin, axis=0, keepdims=True)
        cand = jnp.where(rmin == lmin, ix, jnp.int32(2**30))
        idx_ref[...] = jnp.min(cand, axis=0)
        dsum_ref[...] += jnp.sum(lmin).reshape(1, 1)


def _tc_argmin(z3d, W2, TM=512, TN=8192, GN=128, interpret=False):
    B, K, T = z3d.shape
    M = B * T
    N = W2.shape[0]
    grid = (M // TM, N // TN)
    ti = T // TM
    return pl.pallas_call(
        functools.partial(_argmin_body, TN, GN),
        grid=grid,
        in_specs=[
            pl.BlockSpec((1, K, TM), lambda i, j: (i // ti, 0, i % ti)),
            pl.BlockSpec((N, K), lambda i, j: (0, 0)),
        ],
        out_specs=[
            pl.BlockSpec((TM,), lambda i, j: (i,)),
            pl.BlockSpec((1, 1), lambda i, j: (0, 0)),
        ],
        out_shape=[
            jax.ShapeDtypeStruct((M,), jnp.int32),
            jax.ShapeDtypeStruct((1, 1), jnp.float32),
        ],
        scratch_shapes=[
            pltpu.VMEM((N, 1), jnp.float32),
            pltpu.VMEM((1, TM), jnp.float32),
            pltpu.VMEM((8, TM), jnp.float32),
            pltpu.VMEM((8, TM), jnp.int32),
        ],
        interpret=interpret,
    )(z3d, W2)


@functools.cache
def _sc_gather_hist(M, N, D):
    info = plsc.get_sparse_core_info()
    NC, NS, L = info.num_cores, info.num_subcores, info.num_lanes
    NW = NC * NS
    Bv = M // NW
    CH = 128
    n_ch = Bv // CH
    mesh = plsc.VectorSubcoreMesh(core_axis_name="c", subcore_axis_name="s")

    @functools.partial(
        pl.kernel, mesh=mesh,
        out_type=[jax.ShapeDtypeStruct((M, D), jnp.float32),
                  jax.ShapeDtypeStruct((NC, N), jnp.float32)],
        scratch_types=[
            pltpu.VMEM((Bv,), jnp.int32),
            pltpu.VMEM((CH, D), jnp.float32),
            pltpu.VMEM((CH, D), jnp.float32),
            pltpu.VMEM((Bv,), jnp.float32),
            pltpu.VMEM((N,), jnp.float32),
            pltpu.VMEM_SHARED((N,), jnp.float32),
            pltpu.SemaphoreType.DMA,
            pltpu.SemaphoreType.DMA,
        ],
    )
    def k(w_hbm, idx_hbm, out_hbm, hist_hbm,
          idx_v, bufa, bufb, ones_v, zeros_v, hist_sh, sema, semb):
        c = lax.axis_index("c")
        s = lax.axis_index("s")
        wid = s * NC + c
        base = wid * Bv
        pltpu.sync_copy(idx_hbm.at[pl.ds(base, Bv)], idx_v)

        bufs = [bufa, bufb]
        sems = [sema, semb]
        prev = pltpu.async_copy(w_hbm.at[idx_v.at[pl.ds(0, CH)]], bufa, sema)
        for ch in range(1, n_ch):
            cur = pltpu.async_copy(w_hbm.at[idx_v.at[pl.ds(ch * CH, CH)]],
                                   bufs[ch % 2], sems[ch % 2])
            prev.wait()
            pltpu.sync_copy(bufs[(ch - 1) % 2],
                            out_hbm.at[pl.ds(base + (ch - 1) * CH, CH)])
            prev = cur
        prev.wait()
        pltpu.sync_copy(bufs[(n_ch - 1) % 2],
                        out_hbm.at[pl.ds(base + (n_ch - 1) * CH, CH)])

        def fill_ones(t, carry):
            ones_v[pl.ds(t * L, L)] = jnp.full((L,), 1.0, jnp.float32)
            return carry
        lax.fori_loop(0, Bv // L, fill_ones, 0)

        @pl.when(s == 0)
        def _():
            def fill_zeros(t, carry):
                zeros_v[pl.ds(t * L, L)] = jnp.zeros((L,), jnp.float32)
                return carry
            lax.fori_loop(0, N // L, fill_zeros, 0)
            pltpu.sync_copy(zeros_v, hist_sh)

        plsc.subcore_barrier()
        for kk in range(n_ch):
            pltpu.sync_copy(ones_v.at[pl.ds(kk * CH, CH)],
                            hist_sh.at[idx_v.at[pl.ds(kk * CH, CH)]],
                            add=True)
        plsc.subcore_barrier()

        @pl.when(s == 0)
        def _():
            pltpu.sync_copy(hist_sh, hist_hbm.at[c])

    return k


def kernel(z, W):
    B, C, T = z.shape
    N = W.shape[0]
    n_tok = B * T
    idx, dsum = _tc_argmin(z, W * -2.0)
    quantized, hists = _sc_gather_hist(n_tok, N, C)(W, idx)
    counts = hists[0] + hists[1]
    mse = dsum[0, 0] / (n_tok * C)
    codebook_loss = mse
    commitment_loss = mse
    vq_loss = codebook_loss + _BETA * commitment_loss
    quantized_out = jnp.transpose(quantized.reshape(B, T, C), (0, 2, 1))
    avg_probs = counts / n_tok
    perplexity = jnp.exp(-jnp.sum(avg_probs * jnp.log(avg_probs + 1e-10)))
    return (quantized_out, idx.reshape(B, T), vq_loss, codebook_loss,
            commitment_loss, perplexity)
